# baseline (speedup 1.0000x reference)
import jax
import jax.numpy as jnp
from jax.experimental import pallas as pl

_ROWS = 4608
_BLK = 512
_K = 1024
_V = 32


def _vq_block(x_ref, et_ref, e_ref, out0_ref, out1_ref, out2_ref):
    xb = x_ref[...]
    et = et_ref[...]
    scores = jnp.dot(xb, et, preferred_element_type=jnp.float32,
                     precision=jax.lax.Precision.HIGHEST)
    esq = jnp.sum(et * et, axis=0, keepdims=True)
    dd = esq - 2.0 * scores
    iota = jax.lax.broadcasted_iota(jnp.int32, (_BLK, _K), 1)

    m1 = jnp.min(dd, axis=1, keepdims=True)
    idx1 = jnp.min(jnp.where(dd == m1, iota, _K), axis=1)
    dd2 = jnp.where(iota == idx1[:, None], jnp.inf, dd)
    m2 = jnp.min(dd2, axis=1, keepdims=True)
    idx2 = jnp.min(jnp.where(dd2 == m2, iota, _K), axis=1)

    e_all = e_ref[...]
    oh1 = (iota == idx1[:, None]).astype(jnp.float32)
    oh2 = (iota == idx2[:, None]).astype(jnp.float32)
    ohb = jnp.concatenate([oh1, oh2], axis=0)
    e12 = jnp.dot(ohb, e_all, preferred_element_type=jnp.float32,
                  precision=jax.lax.Precision.HIGHEST)
    e1 = e12[:_BLK]
    e2 = e12[_BLK:]

    s1 = jnp.sum((xb - e1) ** 2, axis=1)
    s2 = jnp.sum((xb - e2) ** 2, axis=1)
    d1 = jnp.sqrt(s1)
    d2 = jnp.sqrt(s2)
    take2 = (d2 < d1) | ((d2 == d1) & (idx2 < idx1))
    outv = jnp.where(take2[:, None], e2, e1)
    dp = jnp.where(take2, d2, d1)

    out0_ref[...] = (outv - xb) + xb
    out1_ref[0, 0, :] = dp * dp
    out2_ref[0, 0, :] = dp * dp


def kernel(x, embedding, offset):
    B, S, C, V = x.shape
    del offset
    x2 = x.reshape(_ROWS, _V)
    e2 = embedding.reshape(_K, _V)
    et = e2.T
    nblk = _ROWS // _BLK
    out0, out1, out2 = pl.pallas_call(
        _vq_block,
        grid=(nblk,),
        in_specs=[
            pl.BlockSpec((_BLK, _V), lambda i: (i, 0)),
            pl.BlockSpec((_V, _K), lambda i: (0, 0)),
            pl.BlockSpec((_K, _V), lambda i: (0, 0)),
        ],
        out_specs=[
            pl.BlockSpec((_BLK, _V), lambda i: (i, 0)),
            pl.BlockSpec((1, 1, _BLK), lambda i: (i, 0, 0)),
            pl.BlockSpec((1, 1, _BLK), lambda i: (i, 0, 0)),
        ],
        out_shape=[
            jax.ShapeDtypeStruct((_ROWS, _V), jnp.float32),
            jax.ShapeDtypeStruct((nblk, 1, _BLK), jnp.float32),
            jax.ShapeDtypeStruct((nblk, 1, _BLK), jnp.float32),
        ],
    )(x2, et, e2)
    return (
        out0.reshape(B, S, C, V),
        out1.reshape(B, S, C),
        out2.reshape(B, S, C),
    )


# speedup vs baseline: 1.6621x; 1.6621x over previous
import jax
import jax.numpy as jnp
from jax.experimental import pallas as pl

_ROWS = 4608
_BLK = 512
_K = 1024
_V = 32


def _split_hi_lo(a):
    hi = a.astype(jnp.bfloat16).astype(jnp.float32)
    lo = a - hi
    return hi, lo


def _vq_block(x_ref, et_ref, e_ref, out0_ref, out1_ref):
    xb = x_ref[...]
    et = et_ref[...]

    xh, xl = _split_hi_lo(xb)
    eth, etl = _split_hi_lo(et)
    xc = jnp.concatenate([xh, xh, xl], axis=1)
    etc = jnp.concatenate([eth, etl, eth], axis=0)
    scores = jnp.dot(xc, etc, preferred_element_type=jnp.float32)

    esq = jnp.sum(et * et, axis=0, keepdims=True)
    dd = esq - 2.0 * scores
    iota = jax.lax.broadcasted_iota(jnp.int32, (_BLK, _K), 1)

    m1 = jnp.min(dd, axis=1, keepdims=True)
    idx1 = jnp.min(jnp.where(dd == m1, iota, _K), axis=1)
    dd2 = jnp.where(iota == idx1[:, None], jnp.inf, dd)
    m2 = jnp.min(dd2, axis=1, keepdims=True)
    idx2 = jnp.min(jnp.where(dd2 == m2, iota, _K), axis=1)

    e_all = e_ref[...]
    e_h = e_all.astype(jnp.bfloat16).astype(jnp.float32)
    e_hm = (e_all - e_h)
    e_m = e_hm.astype(jnp.bfloat16).astype(jnp.float32)
    e_l = e_hm - e_m
    e_parts = jnp.concatenate([e_h, e_m, e_l], axis=1)
    oh1 = (iota == idx1[:, None]).astype(jnp.float32)
    oh2 = (iota == idx2[:, None]).astype(jnp.float32)
    ohb = jnp.concatenate([oh1, oh2], axis=0)
    g = jnp.dot(ohb, e_parts, preferred_element_type=jnp.float32)
    e12 = (g[:, :_V] + g[:, _V:2 * _V]) + g[:, 2 * _V:]
    e1 = e12[:_BLK]
    e2 = e12[_BLK:]

    s1 = jnp.sum((xb - e1) ** 2, axis=1)
    s2 = jnp.sum((xb - e2) ** 2, axis=1)
    d1 = jnp.sqrt(s1)
    d2 = jnp.sqrt(s2)
    take2 = (d2 < d1) | ((d2 == d1) & (idx2 < idx1))
    outv = jnp.where(take2[:, None], e2, e1)
    dp = jnp.where(take2, d2, d1)

    out0_ref[...] = (outv - xb) + xb
    out1_ref[0, 0, :] = dp * dp


def kernel(x, embedding, offset):
    B, S, C, V = x.shape
    del offset
    x2 = x.reshape(_ROWS, _V)
    e2 = embedding.reshape(_K, _V)
    et = e2.T
    nblk = _ROWS // _BLK
    out0, out1 = pl.pallas_call(
        _vq_block,
        grid=(nblk,),
        in_specs=[
            pl.BlockSpec((_BLK, _V), lambda i: (i, 0)),
            pl.BlockSpec((_V, _K), lambda i: (0, 0)),
            pl.BlockSpec((_K, _V), lambda i: (0, 0)),
        ],
        out_specs=[
            pl.BlockSpec((_BLK, _V), lambda i: (i, 0)),
            pl.BlockSpec((1, 1, _BLK), lambda i: (i, 0, 0)),
        ],
        out_shape=[
            jax.ShapeDtypeStruct((_ROWS, _V), jnp.float32),
            jax.ShapeDtypeStruct((nblk, 1, _BLK), jnp.float32),
        ],
    )(x2, et, e2)
    out1 = out1.reshape(B, S, C)
    return (out0.reshape(B, S, C, V), out1, out1)


# speedup vs baseline: 1.9134x; 1.1512x over previous
import jax
import jax.numpy as jnp
from jax.experimental import pallas as pl

_ROWS = 4608
_BLK = 1152
_K = 1024
_V = 32


def _split_hi_lo(a):
    hi = a.astype(jnp.bfloat16).astype(jnp.float32)
    lo = a - hi
    return hi, lo


def _vq_block(x_ref, et_ref, e_ref, out0_ref, out1_ref):
    xb = x_ref[...]
    et = et_ref[...]

    xh, xl = _split_hi_lo(xb)
    eth, etl = _split_hi_lo(et)
    xc = jnp.concatenate([xh, xh, xl], axis=1)
    etc = jnp.concatenate([eth, etl, eth], axis=0)
    scores = jnp.dot(xc, etc, preferred_element_type=jnp.float32)

    esq = jnp.sum(et * et, axis=0, keepdims=True)
    dd = esq - 2.0 * scores
    iota = jax.lax.broadcasted_iota(jnp.int32, (_BLK, _K), 1)

    m1 = jnp.min(dd, axis=1, keepdims=True)
    idx1 = jnp.min(jnp.where(dd == m1, iota, _K), axis=1)
    eq1 = iota == idx1[:, None]
    dd2 = jnp.where(eq1, jnp.inf, dd)
    m2 = jnp.min(dd2, axis=1, keepdims=True)
    idx2 = jnp.min(jnp.where(dd2 == m2, iota, _K), axis=1)
    eq2 = iota == idx2[:, None]

    e_all = e_ref[...]
    e_h = e_all.astype(jnp.bfloat16).astype(jnp.float32)
    e_hm = (e_all - e_h)
    e_m = e_hm.astype(jnp.bfloat16).astype(jnp.float32)
    e_l = e_hm - e_m
    e_parts = jnp.concatenate([e_h, e_m, e_l], axis=1)
    oh1 = eq1.astype(jnp.float32)
    oh2 = eq2.astype(jnp.float32)
    ohb = jnp.concatenate([oh1, oh2], axis=0)
    g = jnp.dot(ohb, e_parts, preferred_element_type=jnp.float32)
    e12 = (g[:, :_V] + g[:, _V:2 * _V]) + g[:, 2 * _V:]
    e1 = e12[:_BLK]
    e2 = e12[_BLK:]

    s1 = jnp.sum((xb - e1) ** 2, axis=1)
    s2 = jnp.sum((xb - e2) ** 2, axis=1)
    d1 = jnp.sqrt(s1)
    d2 = jnp.sqrt(s2)
    take2 = (d2 < d1) | ((d2 == d1) & (idx2 < idx1))
    outv = jnp.where(take2[:, None], e2, e1)
    dp = jnp.where(take2, d2, d1)

    out0_ref[...] = (outv - xb) + xb
    out1_ref[0, 0, :] = dp * dp


def kernel(x, embedding, offset):
    B, S, C, V = x.shape
    del offset
    x2 = x.reshape(_ROWS, _V)
    e2 = embedding.reshape(_K, _V)
    et = e2.T
    nblk = _ROWS // _BLK
    out0, out1 = pl.pallas_call(
        _vq_block,
        grid=(nblk,),
        in_specs=[
            pl.BlockSpec((_BLK, _V), lambda i: (i, 0)),
            pl.BlockSpec((_V, _K), lambda i: (0, 0)),
            pl.BlockSpec((_K, _V), lambda i: (0, 0)),
        ],
        out_specs=[
            pl.BlockSpec((_BLK, _V), lambda i: (i, 0)),
            pl.BlockSpec((1, 1, _BLK), lambda i: (i, 0, 0)),
        ],
        out_shape=[
            jax.ShapeDtypeStruct((_ROWS, _V), jnp.float32),
            jax.ShapeDtypeStruct((nblk, 1, _BLK), jnp.float32),
        ],
    )(x2, et, e2)
    out1 = out1.reshape(B, S, C)
    return (out0.reshape(B, S, C, V), out1, out1)
